# baseline (speedup 1.0000x reference)
import functools

import jax
import jax.numpy as jnp
from jax import lax
from jax.experimental import pallas as pl
from jax.experimental.pallas import tpu as pltpu
from jax.experimental.pallas import tpu_sc as plsc

NC = 2
NS = 16
LANES = 16


def _sc_segment_sum(x, src, dst):
    N, D = x.shape
    E = src.shape[0]
    NW = NC * NS
    e_per_tile = E // NW
    C = 80
    n_iter = e_per_tile // C
    NBUF = 4
    NI = 8
    G = 2
    K = NBUF - G
    n_groups = n_iter // NI
    rows_per_tile = (N // NS) // 8 * 8
    rem_rows = N - rows_per_tile * NS

    mesh = plsc.VectorSubcoreMesh(core_axis_name="c", subcore_axis_name="s")

    @functools.partial(
        pl.kernel,
        out_type=jax.ShapeDtypeStruct((NC, N, D), jnp.float32),
        mesh=mesh,
        scratch_types=[
            [pltpu.VMEM((C, D), jnp.float32) for _ in range(NBUF)],
            [pltpu.VMEM((C,), jnp.int32) for _ in range(NI)],
            [pltpu.VMEM((C,), jnp.int32) for _ in range(NI)],
            pltpu.VMEM_SHARED((N, D), jnp.float32),
            [pltpu.SemaphoreType.DMA for _ in range(NBUF)],
            [pltpu.SemaphoreType.DMA for _ in range(NI)],
            [pltpu.SemaphoreType.DMA for _ in range(NBUF)],
            pltpu.SemaphoreType.DMA,
        ],
    )
    def seg_sum(x_hbm, src_hbm, dst_hbm, out_hbm, rows_v, src_v, dst_v,
                agg_sh, sem_r, sem_i, sem_s, sem_z):
        c = lax.axis_index("c")
        s = lax.axis_index("s")
        wid = c * NS + s
        base = wid * e_per_tile

        def issue_idx(j, bi):
            off = pl.multiple_of(base + j * C, 8)
            pltpu.async_copy(src_hbm.at[pl.ds(off, C)], src_v[bi], sem_i[bi])
            pltpu.async_copy(dst_hbm.at[pl.ds(off, C)], dst_v[bi], sem_i[bi])

        def wait_idx(bi):
            pltpu.make_async_copy(src_hbm.at[pl.ds(0, C)], src_v[bi],
                                  sem_i[bi]).wait()
            pltpu.make_async_copy(dst_hbm.at[pl.ds(0, C)], dst_v[bi],
                                  sem_i[bi]).wait()

        def issue_gather(b, bi):
            pltpu.async_copy(x_hbm.at[src_v[bi]], rows_v[b], sem_r[b])

        def wait_rows(b):
            pltpu.make_async_copy(x_hbm.at[pl.ds(0, C)], rows_v[b],
                                  sem_r[b]).wait()

        def issue_scatter(b, bi):
            pltpu.async_copy(rows_v[b], agg_sh.at[dst_v[bi]], sem_s[b],
                             add=True)

        def wait_scatter(b):
            pltpu.make_async_copy(rows_v[b], agg_sh.at[pl.ds(0, C)],
                                  sem_s[b]).wait()

        for bi in range(NI):
            issue_idx(bi, bi)

        def zrow(r, carry):
            for k in range(D // LANES):
                rows_v[0][r, pl.ds(k * LANES, LANES)] = jnp.zeros(
                    (LANES,), jnp.float32)
            return carry
        lax.fori_loop(0, C, zrow, 0)

        row0 = s * rows_per_tile
        n_full = rows_per_tile // C
        rem = rows_per_tile % C
        for j in range(n_full):
            pltpu.async_copy(rows_v[0], agg_sh.at[pl.ds(row0 + j * C, C)],
                             sem_z)
        if rem:
            pltpu.async_copy(rows_v[0].at[pl.ds(0, rem)],
                             agg_sh.at[pl.ds(row0 + n_full * C, rem)], sem_z)
        if rem_rows:
            @pl.when(s == NS - 1)
            def _():
                pltpu.async_copy(
                    rows_v[0].at[pl.ds(0, rem_rows)],
                    agg_sh.at[pl.ds(NS * rows_per_tile, rem_rows)], sem_z)
        for j in range(n_full):
            pltpu.make_async_copy(rows_v[0], agg_sh.at[pl.ds(0, C)],
                                  sem_z).wait()
        if rem:
            pltpu.make_async_copy(rows_v[0].at[pl.ds(0, rem)],
                                  agg_sh.at[pl.ds(0, rem)], sem_z).wait()
        if rem_rows:
            @pl.when(s == NS - 1)
            def _():
                pltpu.make_async_copy(
                    rows_v[0].at[pl.ds(0, rem_rows)],
                    agg_sh.at[pl.ds(0, rem_rows)], sem_z).wait()

        for b in range(G):
            wait_idx(b)
            issue_gather(b, b)
        plsc.subcore_barrier()

        def chunk_body(j, b, bi):
            bp = (b - K) % NBUF
            bip = (bi - K) % NI
            big = (bi + G) % NI

            @pl.when(j >= K)
            def _retire_prev():
                wait_scatter(bp)

            @pl.when(j + G < n_iter)
            def _next_gather():
                wait_idx(big)
                issue_gather((b + G) % NBUF, big)

            wait_rows(b)
            issue_scatter(b, bi)

            @pl.when((j >= K) & (j + NI - K < n_iter))
            def _refill_idx():
                issue_idx(j - K + NI, bip)

        def group(g, carry):
            j0 = g * NI
            for u in range(NI):
                chunk_body(j0 + u, u % NBUF, u)
            return carry
        lax.fori_loop(0, n_groups, group, 0)
        for j in range(n_groups * NI, n_iter):
            chunk_body(j, j % NBUF, j % NI)
        for j in range(n_iter - K, n_iter):
            wait_scatter(j % NBUF)

        plsc.subcore_barrier()
        pltpu.sync_copy(agg_sh.at[pl.ds(row0, rows_per_tile)],
                        out_hbm.at[c, pl.ds(row0, rows_per_tile)])
        if rem_rows:
            @pl.when(s == NS - 1)
            def _():
                pltpu.sync_copy(
                    agg_sh.at[pl.ds(NS * rows_per_tile, rem_rows)],
                    out_hbm.at[c, pl.ds(NS * rows_per_tile, rem_rows)])

    return seg_sum(x, src, dst)


_BLK = 2000


def _leaky(z):
    return jnp.where(z >= 0.0, z, 0.01 * z)


def _bn_scale_shift(s_ref, q_ref, g_ref, bt_ref, n):
    mean = s_ref[0:1, :] / n
    var = q_ref[0:1, :] / n - mean * mean
    inv = lax.rsqrt(var + 1e-5)
    scale = g_ref[...] * inv
    shift = bt_ref[...] - mean * scale
    return scale, shift


def _gin_layer_body(eps_ref, x_ref, agg_ref, w_ref, b_ref, g_ref, bt_ref,
                    out_ref, lin_ref, s_ref, q_ref, *, n_rows):
    p = pl.program_id(0)
    i = pl.program_id(1)

    @pl.when(p == 0)
    def _():
        h = x_ref[...] * (1.0 + eps_ref[0, 0]) + agg_ref[0] + agg_ref[1]
        lin = jnp.dot(h, w_ref[...], preferred_element_type=jnp.float32)
        lin = lin + b_ref[...]
        lin_ref[pl.ds(i * _BLK, _BLK), :] = lin

        @pl.when(i == 0)
        def _():
            s_ref[...] = jnp.zeros_like(s_ref)
            q_ref[...] = jnp.zeros_like(q_ref)

        s_ref[...] += jnp.broadcast_to(
            jnp.sum(lin, axis=0, keepdims=True), s_ref.shape)
        q_ref[...] += jnp.broadcast_to(
            jnp.sum(lin * lin, axis=0, keepdims=True), q_ref.shape)

    @pl.when(p == 1)
    def _():
        scale, shift = _bn_scale_shift(s_ref, q_ref, g_ref, bt_ref,
                                       float(n_rows))
        y = lin_ref[pl.ds(i * _BLK, _BLK), :] * scale + shift
        out_ref[...] = jnp.where(y >= 0.0, y, 1e-4 * y)


def _gin_layer(eps_arr, x, agg, w, b_row, gamma_row, beta_row):
    N, D = x.shape
    H = w.shape[1]
    grid = N // _BLK
    full = lambda p, i: (0, 0)
    blk = lambda p, i: (i, 0)
    return pl.pallas_call(
        functools.partial(_gin_layer_body, n_rows=N),
        grid=(2, grid),
        in_specs=[
            pl.BlockSpec(memory_space=pltpu.SMEM),
            pl.BlockSpec((_BLK, D), lambda p, i: (i * (1 - p), 0)),
            pl.BlockSpec((NC, _BLK, D), lambda p, i: (0, i * (1 - p), 0)),
            pl.BlockSpec((D, H), full),
            pl.BlockSpec((1, H), full),
            pl.BlockSpec((1, H), full),
            pl.BlockSpec((1, H), full),
        ],
        out_specs=pl.BlockSpec((_BLK, H), blk),
        out_shape=jax.ShapeDtypeStruct((N, H), jnp.float32),
        scratch_shapes=[
            pltpu.VMEM((N, H), jnp.float32),
            pltpu.VMEM((8, H), jnp.float32),
            pltpu.VMEM((8, H), jnp.float32),
        ],
    )(eps_arr, x, agg, w, b_row, gamma_row, beta_row)


def _gin3_head_body(eps_ref, x_ref, agg_ref, w_ref, b_ref, g_ref, bt_ref,
                    w1_ref, b1_ref, w2_ref, b2_ref, w3_ref, b3_ref,
                    wf_ref, bf_ref, out_ref, lin_ref, s_ref, q_ref, *,
                    n_rows):
    p = pl.program_id(0)
    i = pl.program_id(1)

    @pl.when(p == 0)
    def _():
        h = x_ref[...] * (1.0 + eps_ref[0, 0]) + agg_ref[0] + agg_ref[1]
        lin = jnp.dot(h, w_ref[...], preferred_element_type=jnp.float32)
        lin = lin + b_ref[...]
        lin_ref[pl.ds(i * _BLK, _BLK), :] = lin

        @pl.when(i == 0)
        def _():
            s_ref[...] = jnp.zeros_like(s_ref)
            q_ref[...] = jnp.zeros_like(q_ref)

        s_ref[...] += jnp.broadcast_to(
            jnp.sum(lin, axis=0, keepdims=True), s_ref.shape)
        q_ref[...] += jnp.broadcast_to(
            jnp.sum(lin * lin, axis=0, keepdims=True), q_ref.shape)

    @pl.when(p == 1)
    def _():
        scale, shift = _bn_scale_shift(s_ref, q_ref, g_ref, bt_ref,
                                       float(n_rows))
        y = lin_ref[pl.ds(i * _BLK, _BLK), :] * scale + shift
        h = jnp.where(y >= 0.0, y, 1e-4 * y)
        z = jnp.dot(h, w1_ref[...], preferred_element_type=jnp.float32)
        z = z + b1_ref[...]
        z = jnp.dot(z, w2_ref[...], preferred_element_type=jnp.float32)
        z = _leaky(z + b2_ref[...])
        z = jnp.dot(z, w3_ref[...], preferred_element_type=jnp.float32)
        z = _leaky(z + b3_ref[...])
        zf = jnp.sum(z * wf_ref[...], axis=1, keepdims=True)
        zf = zf + bf_ref[0, 0]
        out_ref[...] = jnp.broadcast_to(jax.nn.sigmoid(zf), out_ref.shape)


def _gin3_head(eps_arr, x, agg, w, b_row, gamma_row, beta_row,
               w1, b1, w2, b2, w3, b3, wf_row, bf_arr):
    N, D = x.shape
    H = w.shape[1]
    grid = N // _BLK
    full = lambda p, i: (0, 0)
    return pl.pallas_call(
        functools.partial(_gin3_head_body, n_rows=N),
        grid=(2, grid),
        in_specs=[
            pl.BlockSpec(memory_space=pltpu.SMEM),
            pl.BlockSpec((_BLK, D), lambda p, i: (i * (1 - p), 0)),
            pl.BlockSpec((NC, _BLK, D), lambda p, i: (0, i * (1 - p), 0)),
            pl.BlockSpec((D, H), full),
            pl.BlockSpec((1, H), full),
            pl.BlockSpec((1, H), full),
            pl.BlockSpec((1, H), full),
            pl.BlockSpec((H, H), full), pl.BlockSpec((1, H), full),
            pl.BlockSpec((H, H), full), pl.BlockSpec((1, H), full),
            pl.BlockSpec((H, H), full), pl.BlockSpec((1, H), full),
            pl.BlockSpec((1, H), full),
            pl.BlockSpec(memory_space=pltpu.SMEM),
        ],
        out_specs=pl.BlockSpec((_BLK, H), lambda p, i: (i, 0)),
        out_shape=jax.ShapeDtypeStruct((N, H), jnp.float32),
        scratch_shapes=[
            pltpu.VMEM((N, H), jnp.float32),
            pltpu.VMEM((8, H), jnp.float32),
            pltpu.VMEM((8, H), jnp.float32),
        ],
    )(eps_arr, x, agg, w, b_row, gamma_row, beta_row,
      w1, b1, w2, b2, w3, b3, wf_row, bf_arr)


def kernel(x, edge_index, batch, params):
    N, D = x.shape
    H = params["convs"][0]["W"].shape[1]
    src = edge_index[0]
    dst = edge_index[1]

    cls1 = params["cls1"]
    cls = params["cls"]
    fin = params["final"]
    wf_row = fin["W"].reshape(1, H)
    bf_arr = fin["b"].reshape(1, 1)

    h = x
    for li, layer in enumerate(params["convs"]):
        agg = _sc_segment_sum(h, src, dst)
        eps_arr = layer["eps"].reshape(1, 1)
        args = (eps_arr, h, agg, layer["W"], layer["b"][None, :],
                layer["gamma"][None, :], layer["beta"][None, :])
        if li < 2:
            h = _gin_layer(*args)
        else:
            out = _gin3_head(*args, cls1["W"], cls1["b"][None, :],
                             cls[0]["W"], cls[0]["b"][None, :],
                             cls[1]["W"], cls[1]["b"][None, :],
                             wf_row, bf_arr)
    return out[:, 0:1]


# speedup vs baseline: 1.0851x; 1.0851x over previous
import functools

import jax
import jax.numpy as jnp
from jax import lax
from jax.experimental import pallas as pl
from jax.experimental.pallas import tpu as pltpu
from jax.experimental.pallas import tpu_sc as plsc

NC = 2
NS = 16
LANES = 16


def _sc_segment_sum(x, src, dst):
    N, D = x.shape
    E = src.shape[0]
    NW = NC * NS
    e_per_tile = E // NW
    C = 80
    n_iter = e_per_tile // C
    NBUF = 4
    NI = 8
    G = 3
    K = NBUF - G
    n_groups = n_iter // NI
    rows_per_tile = (N // NS) // 8 * 8
    rem_rows = N - rows_per_tile * NS

    mesh = plsc.VectorSubcoreMesh(core_axis_name="c", subcore_axis_name="s")

    @functools.partial(
        pl.kernel,
        out_type=jax.ShapeDtypeStruct((NC, N, D), jnp.float32),
        mesh=mesh,
        scratch_types=[
            [pltpu.VMEM((C, D), jnp.float32) for _ in range(NBUF)],
            [pltpu.VMEM((C,), jnp.int32) for _ in range(NI)],
            [pltpu.VMEM((C,), jnp.int32) for _ in range(NI)],
            pltpu.VMEM_SHARED((N, D), jnp.float32),
            [pltpu.SemaphoreType.DMA for _ in range(NBUF)],
            [pltpu.SemaphoreType.DMA for _ in range(NI)],
            [pltpu.SemaphoreType.DMA for _ in range(NBUF)],
            pltpu.SemaphoreType.DMA,
        ],
    )
    def seg_sum(x_hbm, src_hbm, dst_hbm, out_hbm, rows_v, src_v, dst_v,
                agg_sh, sem_r, sem_i, sem_s, sem_z):
        c = lax.axis_index("c")
        s = lax.axis_index("s")
        wid = c * NS + s
        base = wid * e_per_tile

        def issue_idx(j, bi):
            off = pl.multiple_of(base + j * C, 8)
            pltpu.async_copy(src_hbm.at[pl.ds(off, C)], src_v[bi], sem_i[bi])
            pltpu.async_copy(dst_hbm.at[pl.ds(off, C)], dst_v[bi], sem_i[bi])

        def wait_idx(bi):
            pltpu.make_async_copy(src_hbm.at[pl.ds(0, C)], src_v[bi],
                                  sem_i[bi]).wait()
            pltpu.make_async_copy(dst_hbm.at[pl.ds(0, C)], dst_v[bi],
                                  sem_i[bi]).wait()

        def issue_gather(b, bi):
            pltpu.async_copy(x_hbm.at[src_v[bi]], rows_v[b], sem_r[b])

        def wait_rows(b):
            pltpu.make_async_copy(x_hbm.at[pl.ds(0, C)], rows_v[b],
                                  sem_r[b]).wait()

        def issue_scatter(b, bi):
            pltpu.async_copy(rows_v[b], agg_sh.at[dst_v[bi]], sem_s[b],
                             add=True)

        def wait_scatter(b):
            pltpu.make_async_copy(rows_v[b], agg_sh.at[pl.ds(0, C)],
                                  sem_s[b]).wait()

        for bi in range(NI):
            issue_idx(bi, bi)

        def zrow(r, carry):
            for k in range(D // LANES):
                rows_v[0][r, pl.ds(k * LANES, LANES)] = jnp.zeros(
                    (LANES,), jnp.float32)
            return carry
        lax.fori_loop(0, C, zrow, 0)

        row0 = s * rows_per_tile
        n_full = rows_per_tile // C
        rem = rows_per_tile % C
        for j in range(n_full):
            pltpu.async_copy(rows_v[0], agg_sh.at[pl.ds(row0 + j * C, C)],
                             sem_z)
        if rem:
            pltpu.async_copy(rows_v[0].at[pl.ds(0, rem)],
                             agg_sh.at[pl.ds(row0 + n_full * C, rem)], sem_z)
        if rem_rows:
            @pl.when(s == NS - 1)
            def _():
                pltpu.async_copy(
                    rows_v[0].at[pl.ds(0, rem_rows)],
                    agg_sh.at[pl.ds(NS * rows_per_tile, rem_rows)], sem_z)
        for j in range(n_full):
            pltpu.make_async_copy(rows_v[0], agg_sh.at[pl.ds(0, C)],
                                  sem_z).wait()
        if rem:
            pltpu.make_async_copy(rows_v[0].at[pl.ds(0, rem)],
                                  agg_sh.at[pl.ds(0, rem)], sem_z).wait()
        if rem_rows:
            @pl.when(s == NS - 1)
            def _():
                pltpu.make_async_copy(
                    rows_v[0].at[pl.ds(0, rem_rows)],
                    agg_sh.at[pl.ds(0, rem_rows)], sem_z).wait()

        for b in range(G):
            wait_idx(b)
            issue_gather(b, b)
        plsc.subcore_barrier()

        def chunk_body(j, b, bi):
            bp = (b - K) % NBUF
            bip = (bi - K) % NI
            big = (bi + G) % NI

            @pl.when(j >= K)
            def _retire_prev():
                wait_scatter(bp)

            @pl.when(j + G < n_iter)
            def _next_gather():
                wait_idx(big)
                issue_gather((b + G) % NBUF, big)

            wait_rows(b)
            issue_scatter(b, bi)

            @pl.when((j >= K) & (j + NI - K < n_iter))
            def _refill_idx():
                issue_idx(j - K + NI, bip)

        def group(g, carry):
            j0 = g * NI
            for u in range(NI):
                chunk_body(j0 + u, u % NBUF, u)
            return carry
        lax.fori_loop(0, n_groups, group, 0)
        for j in range(n_groups * NI, n_iter):
            chunk_body(j, j % NBUF, j % NI)
        for j in range(n_iter - K, n_iter):
            wait_scatter(j % NBUF)

        plsc.subcore_barrier()
        pltpu.sync_copy(agg_sh.at[pl.ds(row0, rows_per_tile)],
                        out_hbm.at[c, pl.ds(row0, rows_per_tile)])
        if rem_rows:
            @pl.when(s == NS - 1)
            def _():
                pltpu.sync_copy(
                    agg_sh.at[pl.ds(NS * rows_per_tile, rem_rows)],
                    out_hbm.at[c, pl.ds(NS * rows_per_tile, rem_rows)])

    return seg_sum(x, src, dst)


_BLK = 2000


def _leaky(z):
    return jnp.where(z >= 0.0, z, 0.01 * z)


def _bn_scale_shift(s_ref, q_ref, g_ref, bt_ref, n):
    mean = s_ref[0:1, :] / n
    var = q_ref[0:1, :] / n - mean * mean
    inv = lax.rsqrt(var + 1e-5)
    scale = g_ref[...] * inv
    shift = bt_ref[...] - mean * scale
    return scale, shift


def _gin_layer_body(eps_ref, x_ref, agg_ref, w_ref, b_ref, g_ref, bt_ref,
                    out_ref, lin_ref, s_ref, q_ref, *, n_rows):
    p = pl.program_id(0)
    i = pl.program_id(1)

    @pl.when(p == 0)
    def _():
        h = x_ref[...] * (1.0 + eps_ref[0, 0]) + agg_ref[0] + agg_ref[1]
        lin = jnp.dot(h, w_ref[...], preferred_element_type=jnp.float32)
        lin = lin + b_ref[...]
        lin_ref[pl.ds(i * _BLK, _BLK), :] = lin

        @pl.when(i == 0)
        def _():
            s_ref[...] = jnp.zeros_like(s_ref)
            q_ref[...] = jnp.zeros_like(q_ref)

        s_ref[...] += jnp.broadcast_to(
            jnp.sum(lin, axis=0, keepdims=True), s_ref.shape)
        q_ref[...] += jnp.broadcast_to(
            jnp.sum(lin * lin, axis=0, keepdims=True), q_ref.shape)

    @pl.when(p == 1)
    def _():
        scale, shift = _bn_scale_shift(s_ref, q_ref, g_ref, bt_ref,
                                       float(n_rows))
        y = lin_ref[pl.ds(i * _BLK, _BLK), :] * scale + shift
        out_ref[...] = jnp.where(y >= 0.0, y, 1e-4 * y)


def _gin_layer(eps_arr, x, agg, w, b_row, gamma_row, beta_row):
    N, D = x.shape
    H = w.shape[1]
    grid = N // _BLK
    full = lambda p, i: (0, 0)
    blk = lambda p, i: (i, 0)
    return pl.pallas_call(
        functools.partial(_gin_layer_body, n_rows=N),
        grid=(2, grid),
        in_specs=[
            pl.BlockSpec(memory_space=pltpu.SMEM),
            pl.BlockSpec((_BLK, D), lambda p, i: (i * (1 - p), 0)),
            pl.BlockSpec((NC, _BLK, D), lambda p, i: (0, i * (1 - p), 0)),
            pl.BlockSpec((D, H), full),
            pl.BlockSpec((1, H), full),
            pl.BlockSpec((1, H), full),
            pl.BlockSpec((1, H), full),
        ],
        out_specs=pl.BlockSpec((_BLK, H), blk),
        out_shape=jax.ShapeDtypeStruct((N, H), jnp.float32),
        scratch_shapes=[
            pltpu.VMEM((N, H), jnp.float32),
            pltpu.VMEM((8, H), jnp.float32),
            pltpu.VMEM((8, H), jnp.float32),
        ],
    )(eps_arr, x, agg, w, b_row, gamma_row, beta_row)


def _gin3_head_body(eps_ref, x_ref, agg_ref, w_ref, b_ref, g_ref, bt_ref,
                    w1_ref, b1_ref, w2_ref, b2_ref, w3_ref, b3_ref,
                    wf_ref, bf_ref, out_ref, lin_ref, s_ref, q_ref, *,
                    n_rows):
    p = pl.program_id(0)
    i = pl.program_id(1)

    @pl.when(p == 0)
    def _():
        h = x_ref[...] * (1.0 + eps_ref[0, 0]) + agg_ref[0] + agg_ref[1]
        lin = jnp.dot(h, w_ref[...], preferred_element_type=jnp.float32)
        lin = lin + b_ref[...]
        lin_ref[pl.ds(i * _BLK, _BLK), :] = lin

        @pl.when(i == 0)
        def _():
            s_ref[...] = jnp.zeros_like(s_ref)
            q_ref[...] = jnp.zeros_like(q_ref)

        s_ref[...] += jnp.broadcast_to(
            jnp.sum(lin, axis=0, keepdims=True), s_ref.shape)
        q_ref[...] += jnp.broadcast_to(
            jnp.sum(lin * lin, axis=0, keepdims=True), q_ref.shape)

    @pl.when(p == 1)
    def _():
        scale, shift = _bn_scale_shift(s_ref, q_ref, g_ref, bt_ref,
                                       float(n_rows))
        y = lin_ref[pl.ds(i * _BLK, _BLK), :] * scale + shift
        h = jnp.where(y >= 0.0, y, 1e-4 * y)
        z = jnp.dot(h, w1_ref[...], preferred_element_type=jnp.float32)
        z = z + b1_ref[...]
        z = jnp.dot(z, w2_ref[...], preferred_element_type=jnp.float32)
        z = _leaky(z + b2_ref[...])
        z = jnp.dot(z, w3_ref[...], preferred_element_type=jnp.float32)
        z = _leaky(z + b3_ref[...])
        zf = jnp.sum(z * wf_ref[...], axis=1, keepdims=True)
        zf = zf + bf_ref[0, 0]
        out_ref[...] = jnp.broadcast_to(jax.nn.sigmoid(zf), out_ref.shape)


def _gin3_head(eps_arr, x, agg, w, b_row, gamma_row, beta_row,
               w1, b1, w2, b2, w3, b3, wf_row, bf_arr):
    N, D = x.shape
    H = w.shape[1]
    grid = N // _BLK
    full = lambda p, i: (0, 0)
    return pl.pallas_call(
        functools.partial(_gin3_head_body, n_rows=N),
        grid=(2, grid),
        in_specs=[
            pl.BlockSpec(memory_space=pltpu.SMEM),
            pl.BlockSpec((_BLK, D), lambda p, i: (i * (1 - p), 0)),
            pl.BlockSpec((NC, _BLK, D), lambda p, i: (0, i * (1 - p), 0)),
            pl.BlockSpec((D, H), full),
            pl.BlockSpec((1, H), full),
            pl.BlockSpec((1, H), full),
            pl.BlockSpec((1, H), full),
            pl.BlockSpec((H, H), full), pl.BlockSpec((1, H), full),
            pl.BlockSpec((H, H), full), pl.BlockSpec((1, H), full),
            pl.BlockSpec((H, H), full), pl.BlockSpec((1, H), full),
            pl.BlockSpec((1, H), full),
            pl.BlockSpec(memory_space=pltpu.SMEM),
        ],
        out_specs=pl.BlockSpec((_BLK, H), lambda p, i: (i, 0)),
        out_shape=jax.ShapeDtypeStruct((N, H), jnp.float32),
        scratch_shapes=[
            pltpu.VMEM((N, H), jnp.float32),
            pltpu.VMEM((8, H), jnp.float32),
            pltpu.VMEM((8, H), jnp.float32),
        ],
    )(eps_arr, x, agg, w, b_row, gamma_row, beta_row,
      w1, b1, w2, b2, w3, b3, wf_row, bf_arr)


def kernel(x, edge_index, batch, params):
    N, D = x.shape
    H = params["convs"][0]["W"].shape[1]
    src = edge_index[0]
    dst = edge_index[1]

    cls1 = params["cls1"]
    cls = params["cls"]
    fin = params["final"]
    wf_row = fin["W"].reshape(1, H)
    bf_arr = fin["b"].reshape(1, 1)

    h = x
    for li, layer in enumerate(params["convs"]):
        agg = _sc_segment_sum(h, src, dst)
        eps_arr = layer["eps"].reshape(1, 1)
        args = (eps_arr, h, agg, layer["W"], layer["b"][None, :],
                layer["gamma"][None, :], layer["beta"][None, :])
        if li < 2:
            h = _gin_layer(*args)
        else:
            out = _gin3_head(*args, cls1["W"], cls1["b"][None, :],
                             cls[0]["W"], cls[0]["b"][None, :],
                             cls[1]["W"], cls[1]["b"][None, :],
                             wf_row, bf_arr)
    return out[:, 0:1]
